# Initial kernel scaffold; baseline (speedup 1.0000x reference)
import functools

import jax
import jax.numpy as jnp
from jax import lax
from jax.experimental import pallas as pl
from jax.experimental.pallas import tpu as pltpu
from jax.experimental.pallas import tpu_sc as plsc

N = 10000
E = 320000
D_IN = 128
D_H = 64
NG = 64

NC = 2
NS = 16
L = 16
NW = NC * NS
CHUNK = 128
NCHUNK = 80
E_PAD = NW * NCHUNK * CHUNK
N_PAD = 10240
RPT = N_PAD // NS
BLK = 1024

_mesh = plsc.VectorSubcoreMesh(core_axis_name="c", subcore_axis_name="s")


@functools.partial(
    pl.kernel,
    out_type=jax.ShapeDtypeStruct((NC, N_PAD), jnp.float32),
    mesh=_mesh,
    scratch_types=[
        pltpu.VMEM_SHARED((N_PAD,), jnp.float32),
        pltpu.VMEM((NCHUNK, CHUNK), jnp.int32),
        pltpu.VMEM((CHUNK,), jnp.float32),
        pltpu.VMEM((RPT,), jnp.float32),
    ],
)
def _sc_degree(dst_hbm, deg_out, deg_sh, idx_v, ones_v, zero_v):
    c = lax.axis_index("c")
    s = lax.axis_index("s")
    wid = c * NS + s
    for i in range(CHUNK // L):
        ones_v[pl.ds(i * L, L)] = jnp.full((L,), 1.0, jnp.float32)
    for i in range(RPT // L):
        zero_v[pl.ds(i * L, L)] = jnp.zeros((L,), jnp.float32)
    pltpu.sync_copy(zero_v, deg_sh.at[pl.ds(s * RPT, RPT)])
    pltpu.sync_copy(dst_hbm.at[wid], idx_v)
    plsc.subcore_barrier()

    def body(j, carry):
        pltpu.sync_copy(ones_v, deg_sh.at[idx_v.at[j]], add=True)
        return carry

    lax.fori_loop(0, NCHUNK, body, 0)
    plsc.subcore_barrier()
    pltpu.sync_copy(deg_sh.at[pl.ds(s * RPT, RPT)],
                    deg_out.at[c, pl.ds(s * RPT, RPT)])


@functools.partial(
    pl.kernel,
    out_type=jax.ShapeDtypeStruct((NC, N_PAD, D_H), jnp.float32),
    mesh=_mesh,
    scratch_types=[
        pltpu.VMEM_SHARED((N_PAD, D_H), jnp.float32),
        pltpu.VMEM((NCHUNK, CHUNK), jnp.int32),
        pltpu.VMEM((NCHUNK, CHUNK), jnp.int32),
        pltpu.VMEM((CHUNK, D_H), jnp.float32),
        pltpu.VMEM((64, D_H), jnp.float32),
        pltpu.SemaphoreType.DMA,
    ],
)
def _sc_scatter(g_hbm, src_hbm, dst_hbm, acc_out,
                acc_sh, src_v, dst_v, rows_v, zero_v, sem):
    c = lax.axis_index("c")
    s = lax.axis_index("s")
    wid = c * NS + s
    for i in range(64):
        for k in range(D_H // L):
            zero_v[i, pl.ds(k * L, L)] = jnp.zeros((L,), jnp.float32)
    for k in range(RPT // 64):
        pltpu.sync_copy(zero_v, acc_sh.at[pl.ds(s * RPT + k * 64, 64)])
    pltpu.sync_copy(src_hbm.at[wid], src_v)
    pltpu.sync_copy(dst_hbm.at[wid], dst_v)
    plsc.subcore_barrier()

    def body(j, carry):
        pltpu.async_copy(g_hbm.at[src_v.at[j]], rows_v, sem).wait()
        pltpu.sync_copy(rows_v, acc_sh.at[dst_v.at[j]], add=True)
        return carry

    lax.fori_loop(0, NCHUNK, body, 0)
    plsc.subcore_barrier()
    pltpu.sync_copy(acc_sh.at[pl.ds(s * RPT, RPT)],
                    acc_out.at[c, pl.ds(s * RPT, RPT)])


def _mm_body(x_ref, w_ref, o_ref):
    o_ref[...] = jnp.dot(x_ref[...], w_ref[...], preferred_element_type=jnp.float32)


def _matmul(x, w):
    m, k = x.shape
    n = w.shape[1]
    return pl.pallas_call(
        _mm_body,
        grid=(m // BLK,),
        in_specs=[pl.BlockSpec((BLK, k), lambda i: (i, 0)),
                  pl.BlockSpec((k, n), lambda i: (0, 0))],
        out_specs=pl.BlockSpec((BLK, n), lambda i: (i, 0)),
        out_shape=jax.ShapeDtypeStruct((m, n), jnp.float32),
    )(x, w)


def _scale_body(h_ref, deg_ref, g_ref, dinv_ref):
    dv = lax.rsqrt(deg_ref[0] + deg_ref[1] + 1.0)
    dinv_ref[...] = dv
    g_ref[...] = h_ref[...] * dv


def _scale(h, deg_part):
    return pl.pallas_call(
        _scale_body,
        grid=(N_PAD // BLK,),
        in_specs=[pl.BlockSpec((BLK, D_H), lambda i: (i, 0)),
                  pl.BlockSpec((NC, BLK, 1), lambda i: (0, i, 0))],
        out_specs=[pl.BlockSpec((BLK, D_H), lambda i: (i, 0)),
                   pl.BlockSpec((BLK, 1), lambda i: (i, 0))],
        out_shape=[jax.ShapeDtypeStruct((N_PAD, D_H), jnp.float32),
                   jax.ShapeDtypeStruct((N_PAD, 1), jnp.float32)],
    )(h, deg_part)


def _layer2_body(acc_ref, g_ref, dinv_ref, b_ref, w_ref, o_ref):
    dv = dinv_ref[...]
    h1 = jnp.maximum(dv * (acc_ref[0] + acc_ref[1] + g_ref[...]) + b_ref[...], 0.0)
    o_ref[...] = jnp.dot(h1, w_ref[...], preferred_element_type=jnp.float32) * dv


def _layer2(acc, g1, dinv, b1, w2):
    return pl.pallas_call(
        _layer2_body,
        grid=(N_PAD // BLK,),
        in_specs=[pl.BlockSpec((NC, BLK, D_H), lambda i: (0, i, 0)),
                  pl.BlockSpec((BLK, D_H), lambda i: (i, 0)),
                  pl.BlockSpec((BLK, 1), lambda i: (i, 0)),
                  pl.BlockSpec((1, D_H), lambda i: (0, 0)),
                  pl.BlockSpec((D_H, D_H), lambda i: (0, 0))],
        out_specs=pl.BlockSpec((BLK, D_H), lambda i: (i, 0)),
        out_shape=jax.ShapeDtypeStruct((N_PAD, D_H), jnp.float32),
    )(acc, g1, dinv, b1, w2)


def _pool_body(acc_ref, g_ref, dinv_ref, b_ref, batch_ref, wl_ref, bl_ref,
               o_ref, sums_scr, cnt_scr):
    i = pl.program_id(0)

    @pl.when(i == 0)
    def _():
        sums_scr[...] = jnp.zeros_like(sums_scr)
        cnt_scr[...] = jnp.zeros_like(cnt_scr)

    dv = dinv_ref[...]
    h2 = jnp.maximum(dv * (acc_ref[0] + acc_ref[1] + g_ref[...]) + b_ref[...], 0.0)
    p = (batch_ref[...] == lax.broadcasted_iota(jnp.int32, (BLK, NG), 1))
    p = p.astype(jnp.float32)
    sums_scr[...] += lax.dot_general(p, h2, (((0,), (0,)), ((), ())),
                                     preferred_element_type=jnp.float32)
    cnt_scr[...] += lax.dot_general(p, jnp.ones((BLK, 1), jnp.float32),
                                    (((0,), (0,)), ((), ())),
                                    preferred_element_type=jnp.float32)

    @pl.when(i == pl.num_programs(0) - 1)
    def _():
        pooled = sums_scr[...] / jnp.maximum(cnt_scr[...], 1.0)
        o_ref[...] = jnp.dot(pooled, wl_ref[...],
                             preferred_element_type=jnp.float32) + bl_ref[...]


def _pool(acc, g2, dinv, b2, batch_pad, wlin, blin):
    return pl.pallas_call(
        _pool_body,
        grid=(N_PAD // BLK,),
        in_specs=[pl.BlockSpec((NC, BLK, D_H), lambda i: (0, i, 0)),
                  pl.BlockSpec((BLK, D_H), lambda i: (i, 0)),
                  pl.BlockSpec((BLK, 1), lambda i: (i, 0)),
                  pl.BlockSpec((1, D_H), lambda i: (0, 0)),
                  pl.BlockSpec((BLK, 1), lambda i: (i, 0)),
                  pl.BlockSpec((D_H, 2), lambda i: (0, 0)),
                  pl.BlockSpec((1, 2), lambda i: (0, 0))],
        out_specs=pl.BlockSpec((NG, 2), lambda i: (0, 0)),
        out_shape=jax.ShapeDtypeStruct((NG, 2), jnp.float32),
        scratch_shapes=[pltpu.VMEM((NG, D_H), jnp.float32),
                        pltpu.VMEM((NG, 1), jnp.float32)],
    )(acc, g2, dinv, b2, batch_pad, wlin, blin)


def kernel(x, edge_index, batch, W1, b1, W2, b2, Wlin, blin):
    src = edge_index[0].astype(jnp.int32)
    dst = edge_index[1].astype(jnp.int32)
    pad_e = E_PAD - E
    pad_ids = jnp.full((pad_e,), N, jnp.int32)
    src_l = jnp.concatenate([src, pad_ids]).reshape(NW, NCHUNK, CHUNK)
    dst_l = jnp.concatenate([dst, pad_ids]).reshape(NW, NCHUNK, CHUNK)
    x_pad = jnp.concatenate([x, jnp.zeros((N_PAD - N, D_IN), x.dtype)])
    batch_pad = jnp.concatenate(
        [batch.astype(jnp.int32), jnp.full((N_PAD - N,), NG, jnp.int32)]
    ).reshape(N_PAD, 1)

    deg_part = _sc_degree(dst_l)
    h1 = _matmul(x_pad, W1)
    g1, dinv = _scale(h1, deg_part.reshape(NC, N_PAD, 1))
    acc1 = _sc_scatter(g1, src_l, dst_l)
    g2 = _layer2(acc1, g1, dinv, b1.reshape(1, D_H), W2)
    acc2 = _sc_scatter(g2, src_l, dst_l)
    return _pool(acc2, g2, dinv, b2.reshape(1, D_H), batch_pad,
                 Wlin, blin.reshape(1, 2))


# speedup vs baseline: 16.2947x; 16.2947x over previous
import functools

import jax
import jax.numpy as jnp
from jax import lax
from jax.experimental import pallas as pl
from jax.experimental.pallas import tpu as pltpu
from jax.experimental.pallas import tpu_sc as plsc

N = 10000
E = 320000
D_IN = 128
D_H = 64
NG = 64

NC = 2
NS = 16
L = 16
NW = NC * NS
CHUNK = 128
NCHUNK = 80
E_PAD = NW * NCHUNK * CHUNK
N_PAD = 10240
RPT = N_PAD // NS
BLK = 1024

_mesh = plsc.VectorSubcoreMesh(core_axis_name="c", subcore_axis_name="s")
_sc_params = pltpu.CompilerParams(use_tc_tiling_on_sc=False)


@functools.partial(
    pl.kernel,
    out_type=jax.ShapeDtypeStruct((NC, N_PAD), jnp.float32),
    mesh=_mesh,
    scratch_types=[
        pltpu.VMEM_SHARED((N_PAD,), jnp.float32),
        pltpu.VMEM((NCHUNK, CHUNK), jnp.int32),
        pltpu.VMEM((CHUNK,), jnp.float32),
        pltpu.VMEM((RPT,), jnp.float32),
    ],
    compiler_params=_sc_params,
)
def _sc_degree(dst_hbm, deg_out, deg_sh, idx_v, ones_v, zero_v):
    c = lax.axis_index("c")
    s = lax.axis_index("s")
    wid = c * NS + s
    for i in range(CHUNK // L):
        ones_v[pl.ds(i * L, L)] = jnp.full((L,), 1.0, jnp.float32)
    for i in range(RPT // L):
        zero_v[pl.ds(i * L, L)] = jnp.zeros((L,), jnp.float32)
    pltpu.sync_copy(zero_v, deg_sh.at[pl.ds(s * RPT, RPT)])
    pltpu.sync_copy(dst_hbm.at[wid], idx_v)
    plsc.subcore_barrier()

    def body(j, carry):
        pltpu.sync_copy(ones_v, deg_sh.at[idx_v.at[j]], add=True)
        return carry

    lax.fori_loop(0, NCHUNK, body, 0)
    plsc.subcore_barrier()
    pltpu.sync_copy(deg_sh.at[pl.ds(s * RPT, RPT)],
                    deg_out.at[c, pl.ds(s * RPT, RPT)])


@functools.partial(
    pl.kernel,
    out_type=jax.ShapeDtypeStruct((NC, N_PAD, D_H), jnp.float32),
    mesh=_mesh,
    scratch_types=[
        pltpu.VMEM_SHARED((N_PAD, D_H), jnp.float32),
        pltpu.VMEM((NCHUNK, CHUNK), jnp.int32),
        pltpu.VMEM((NCHUNK, CHUNK), jnp.int32),
        pltpu.VMEM((CHUNK, D_H), jnp.float32),
        pltpu.VMEM((64, D_H), jnp.float32),
        pltpu.SemaphoreType.DMA,
    ],
    compiler_params=_sc_params,
)
def _sc_scatter(g_hbm, src_hbm, dst_hbm, acc_out,
                acc_sh, src_v, dst_v, rows_v, zero_v, sem):
    c = lax.axis_index("c")
    s = lax.axis_index("s")
    wid = c * NS + s
    for i in range(64):
        for k in range(D_H // L):
            zero_v[i, pl.ds(k * L, L)] = jnp.zeros((L,), jnp.float32)
    for k in range(RPT // 64):
        pltpu.sync_copy(zero_v, acc_sh.at[pl.ds(s * RPT + k * 64, 64)])
    pltpu.sync_copy(src_hbm.at[wid], src_v)
    pltpu.sync_copy(dst_hbm.at[wid], dst_v)
    plsc.subcore_barrier()

    def body(j, carry):
        pltpu.async_copy(g_hbm.at[src_v.at[j]], rows_v, sem).wait()
        pltpu.sync_copy(rows_v, acc_sh.at[dst_v.at[j]], add=True)
        return carry

    lax.fori_loop(0, NCHUNK, body, 0)
    plsc.subcore_barrier()
    pltpu.sync_copy(acc_sh.at[pl.ds(s * RPT, RPT)],
                    acc_out.at[c, pl.ds(s * RPT, RPT)])


def _mm_body(x_ref, w_ref, o_ref):
    o_ref[...] = jnp.dot(x_ref[...], w_ref[...], preferred_element_type=jnp.float32)


def _matmul(x, w):
    m, k = x.shape
    n = w.shape[1]
    return pl.pallas_call(
        _mm_body,
        grid=(m // BLK,),
        in_specs=[pl.BlockSpec((BLK, k), lambda i: (i, 0)),
                  pl.BlockSpec((k, n), lambda i: (0, 0))],
        out_specs=pl.BlockSpec((BLK, n), lambda i: (i, 0)),
        out_shape=jax.ShapeDtypeStruct((m, n), jnp.float32),
    )(x, w)


def _scale_body(h_ref, deg_ref, g_ref, dinv_ref):
    dv = lax.rsqrt(deg_ref[0] + deg_ref[1] + 1.0)
    dinv_ref[...] = dv
    g_ref[...] = h_ref[...] * dv


def _scale(h, deg_part):
    return pl.pallas_call(
        _scale_body,
        grid=(N_PAD // BLK,),
        in_specs=[pl.BlockSpec((BLK, D_H), lambda i: (i, 0)),
                  pl.BlockSpec((NC, BLK, 1), lambda i: (0, i, 0))],
        out_specs=[pl.BlockSpec((BLK, D_H), lambda i: (i, 0)),
                   pl.BlockSpec((BLK, 1), lambda i: (i, 0))],
        out_shape=[jax.ShapeDtypeStruct((N_PAD, D_H), jnp.float32),
                   jax.ShapeDtypeStruct((N_PAD, 1), jnp.float32)],
    )(h, deg_part)


def _layer2_body(acc_ref, g_ref, dinv_ref, b_ref, w_ref, o_ref):
    dv = dinv_ref[...]
    h1 = jnp.maximum(dv * (acc_ref[0] + acc_ref[1] + g_ref[...]) + b_ref[...], 0.0)
    o_ref[...] = jnp.dot(h1, w_ref[...], preferred_element_type=jnp.float32) * dv


def _layer2(acc, g1, dinv, b1, w2):
    return pl.pallas_call(
        _layer2_body,
        grid=(N_PAD // BLK,),
        in_specs=[pl.BlockSpec((NC, BLK, D_H), lambda i: (0, i, 0)),
                  pl.BlockSpec((BLK, D_H), lambda i: (i, 0)),
                  pl.BlockSpec((BLK, 1), lambda i: (i, 0)),
                  pl.BlockSpec((1, D_H), lambda i: (0, 0)),
                  pl.BlockSpec((D_H, D_H), lambda i: (0, 0))],
        out_specs=pl.BlockSpec((BLK, D_H), lambda i: (i, 0)),
        out_shape=jax.ShapeDtypeStruct((N_PAD, D_H), jnp.float32),
    )(acc, g1, dinv, b1, w2)


def _pool_body(acc_ref, g_ref, dinv_ref, b_ref, batch_ref, wl_ref, bl_ref,
               o_ref, sums_scr, cnt_scr):
    i = pl.program_id(0)

    @pl.when(i == 0)
    def _():
        sums_scr[...] = jnp.zeros_like(sums_scr)
        cnt_scr[...] = jnp.zeros_like(cnt_scr)

    dv = dinv_ref[...]
    h2 = jnp.maximum(dv * (acc_ref[0] + acc_ref[1] + g_ref[...]) + b_ref[...], 0.0)
    p = (batch_ref[...] == lax.broadcasted_iota(jnp.int32, (BLK, NG), 1))
    p = p.astype(jnp.float32)
    sums_scr[...] += lax.dot_general(p, h2, (((0,), (0,)), ((), ())),
                                     preferred_element_type=jnp.float32)
    cnt_scr[...] += lax.dot_general(p, jnp.ones((BLK, 1), jnp.float32),
                                    (((0,), (0,)), ((), ())),
                                    preferred_element_type=jnp.float32)

    @pl.when(i == pl.num_programs(0) - 1)
    def _():
        pooled = sums_scr[...] / jnp.maximum(cnt_scr[...], 1.0)
        o_ref[...] = jnp.dot(pooled, wl_ref[...],
                             preferred_element_type=jnp.float32) + bl_ref[...]


def _pool(acc, g2, dinv, b2, batch_pad, wlin, blin):
    return pl.pallas_call(
        _pool_body,
        grid=(N_PAD // BLK,),
        in_specs=[pl.BlockSpec((NC, BLK, D_H), lambda i: (0, i, 0)),
                  pl.BlockSpec((BLK, D_H), lambda i: (i, 0)),
                  pl.BlockSpec((BLK, 1), lambda i: (i, 0)),
                  pl.BlockSpec((1, D_H), lambda i: (0, 0)),
                  pl.BlockSpec((BLK, 1), lambda i: (i, 0)),
                  pl.BlockSpec((D_H, 2), lambda i: (0, 0)),
                  pl.BlockSpec((1, 2), lambda i: (0, 0))],
        out_specs=pl.BlockSpec((NG, 2), lambda i: (0, 0)),
        out_shape=jax.ShapeDtypeStruct((NG, 2), jnp.float32),
        scratch_shapes=[pltpu.VMEM((NG, D_H), jnp.float32),
                        pltpu.VMEM((NG, 1), jnp.float32)],
    )(acc, g2, dinv, b2, batch_pad, wlin, blin)


def kernel(x, edge_index, batch, W1, b1, W2, b2, Wlin, blin):
    src = edge_index[0].astype(jnp.int32)
    dst = edge_index[1].astype(jnp.int32)
    pad_e = E_PAD - E
    pad_ids = jnp.full((pad_e,), N, jnp.int32)
    src_l = jnp.concatenate([src, pad_ids]).reshape(NW, NCHUNK, CHUNK)
    dst_l = jnp.concatenate([dst, pad_ids]).reshape(NW, NCHUNK, CHUNK)
    x_pad = jnp.concatenate([x, jnp.zeros((N_PAD - N, D_IN), x.dtype)])
    batch_pad = jnp.concatenate(
        [batch.astype(jnp.int32), jnp.full((N_PAD - N,), NG, jnp.int32)]
    ).reshape(N_PAD, 1)

    deg_part = _sc_degree(dst_l)
    h1 = _matmul(x_pad, W1)
    g1, dinv = _scale(h1, deg_part.reshape(NC, N_PAD, 1))
    acc1 = _sc_scatter(g1, src_l, dst_l)
    g2 = _layer2(acc1, g1, dinv, b1.reshape(1, D_H), W2)
    acc2 = _sc_scatter(g2, src_l, dst_l)
    return _pool(acc2, g2, dinv, b2.reshape(1, D_H), batch_pad,
                 Wlin, blin.reshape(1, 2))


# speedup vs baseline: 19.5158x; 1.1977x over previous
import functools

import jax
import jax.numpy as jnp
from jax import lax
from jax.experimental import pallas as pl
from jax.experimental.pallas import tpu as pltpu
from jax.experimental.pallas import tpu_sc as plsc

N = 10000
E = 320000
D_IN = 128
D_H = 64
NG = 64

NC = 2
NS = 16
L = 16
NW = NC * NS
CHUNK = 128
NCHUNK = 80
E_PAD = NW * NCHUNK * CHUNK
N_PAD = 10240
RPT = N_PAD // NS
BLK = 1024

_mesh = plsc.VectorSubcoreMesh(core_axis_name="c", subcore_axis_name="s")
_sc_params = pltpu.CompilerParams(use_tc_tiling_on_sc=False)


@functools.partial(
    pl.kernel,
    out_type=jax.ShapeDtypeStruct((NC, N_PAD), jnp.float32),
    mesh=_mesh,
    scratch_types=[
        pltpu.VMEM_SHARED((N_PAD,), jnp.float32),
        pltpu.VMEM((NCHUNK, CHUNK), jnp.int32),
        pltpu.VMEM((CHUNK,), jnp.float32),
        pltpu.VMEM((RPT,), jnp.float32),
    ],
    compiler_params=_sc_params,
)
def _sc_degree(dst_hbm, deg_out, deg_sh, idx_v, ones_v, zero_v):
    c = lax.axis_index("c")
    s = lax.axis_index("s")
    wid = c * NS + s
    for i in range(CHUNK // L):
        ones_v[pl.ds(i * L, L)] = jnp.full((L,), 1.0, jnp.float32)
    for i in range(RPT // L):
        zero_v[pl.ds(i * L, L)] = jnp.zeros((L,), jnp.float32)
    pltpu.sync_copy(zero_v, deg_sh.at[pl.ds(s * RPT, RPT)])
    pltpu.sync_copy(dst_hbm.at[wid], idx_v)
    plsc.subcore_barrier()

    def body(j, carry):
        pltpu.sync_copy(ones_v, deg_sh.at[idx_v.at[j]], add=True)
        return carry

    lax.fori_loop(0, NCHUNK, body, 0)
    plsc.subcore_barrier()
    pltpu.sync_copy(deg_sh.at[pl.ds(s * RPT, RPT)],
                    deg_out.at[c, pl.ds(s * RPT, RPT)])


@functools.partial(
    pl.kernel,
    out_type=jax.ShapeDtypeStruct((NC, N_PAD, D_H), jnp.float32),
    mesh=_mesh,
    scratch_types=[
        pltpu.VMEM_SHARED((N_PAD, D_H), jnp.float32),
        pltpu.VMEM((NCHUNK, CHUNK), jnp.int32),
        pltpu.VMEM((NCHUNK, CHUNK), jnp.int32),
        pltpu.VMEM((CHUNK, D_H), jnp.float32),
        pltpu.VMEM((CHUNK, D_H), jnp.float32),
        pltpu.VMEM((64, D_H), jnp.float32),
        pltpu.SemaphoreType.DMA,
        pltpu.SemaphoreType.DMA,
    ],
    compiler_params=_sc_params,
)
def _sc_scatter(g_hbm, src_hbm, dst_hbm, acc_out,
                acc_sh, src_v, dst_v, rows0, rows1, zero_v, sem0, sem1):
    c = lax.axis_index("c")
    s = lax.axis_index("s")
    wid = c * NS + s
    for i in range(64):
        for k in range(D_H // L):
            zero_v[i, pl.ds(k * L, L)] = jnp.zeros((L,), jnp.float32)
    for k in range(RPT // 64):
        pltpu.sync_copy(zero_v, acc_sh.at[pl.ds(s * RPT + k * 64, 64)])
    pltpu.sync_copy(src_hbm.at[wid], src_v)
    pltpu.sync_copy(dst_hbm.at[wid], dst_v)
    plsc.subcore_barrier()
    pltpu.async_copy(g_hbm.at[src_v.at[0]], rows0, sem0)

    def body(t, carry):
        c0 = 2 * t
        pltpu.async_copy(g_hbm.at[src_v.at[c0 + 1]], rows1, sem1)
        pltpu.make_async_copy(g_hbm.at[src_v.at[c0]], rows0, sem0).wait()
        pltpu.sync_copy(rows0, acc_sh.at[dst_v.at[c0]], add=True)
        pltpu.async_copy(g_hbm.at[src_v.at[lax.rem(c0 + 2, NCHUNK)]], rows0, sem0)
        pltpu.make_async_copy(g_hbm.at[src_v.at[c0 + 1]], rows1, sem1).wait()
        pltpu.sync_copy(rows1, acc_sh.at[dst_v.at[c0 + 1]], add=True)
        return carry

    lax.fori_loop(0, NCHUNK // 2, body, 0)
    pltpu.make_async_copy(g_hbm.at[src_v.at[0]], rows0, sem0).wait()
    plsc.subcore_barrier()
    pltpu.sync_copy(acc_sh.at[pl.ds(s * RPT, RPT)],
                    acc_out.at[c, pl.ds(s * RPT, RPT)])


def _mm_body(x_ref, w_ref, o_ref):
    o_ref[...] = jnp.dot(x_ref[...], w_ref[...], preferred_element_type=jnp.float32)


def _matmul(x, w):
    m, k = x.shape
    n = w.shape[1]
    return pl.pallas_call(
        _mm_body,
        grid=(m // BLK,),
        in_specs=[pl.BlockSpec((BLK, k), lambda i: (i, 0)),
                  pl.BlockSpec((k, n), lambda i: (0, 0))],
        out_specs=pl.BlockSpec((BLK, n), lambda i: (i, 0)),
        out_shape=jax.ShapeDtypeStruct((m, n), jnp.float32),
    )(x, w)


def _scale_body(h_ref, deg_ref, g_ref, dinv_ref):
    dv = lax.rsqrt(deg_ref[0] + deg_ref[1] + 1.0)
    dinv_ref[...] = dv
    g_ref[...] = h_ref[...] * dv


def _scale(h, deg_part):
    return pl.pallas_call(
        _scale_body,
        grid=(N_PAD // BLK,),
        in_specs=[pl.BlockSpec((BLK, D_H), lambda i: (i, 0)),
                  pl.BlockSpec((NC, BLK, 1), lambda i: (0, i, 0))],
        out_specs=[pl.BlockSpec((BLK, D_H), lambda i: (i, 0)),
                   pl.BlockSpec((BLK, 1), lambda i: (i, 0))],
        out_shape=[jax.ShapeDtypeStruct((N_PAD, D_H), jnp.float32),
                   jax.ShapeDtypeStruct((N_PAD, 1), jnp.float32)],
    )(h, deg_part)


def _layer2_body(acc_ref, g_ref, dinv_ref, b_ref, w_ref, o_ref):
    dv = dinv_ref[...]
    h1 = jnp.maximum(dv * (acc_ref[0] + acc_ref[1] + g_ref[...]) + b_ref[...], 0.0)
    o_ref[...] = jnp.dot(h1, w_ref[...], preferred_element_type=jnp.float32) * dv


def _layer2(acc, g1, dinv, b1, w2):
    return pl.pallas_call(
        _layer2_body,
        grid=(N_PAD // BLK,),
        in_specs=[pl.BlockSpec((NC, BLK, D_H), lambda i: (0, i, 0)),
                  pl.BlockSpec((BLK, D_H), lambda i: (i, 0)),
                  pl.BlockSpec((BLK, 1), lambda i: (i, 0)),
                  pl.BlockSpec((1, D_H), lambda i: (0, 0)),
                  pl.BlockSpec((D_H, D_H), lambda i: (0, 0))],
        out_specs=pl.BlockSpec((BLK, D_H), lambda i: (i, 0)),
        out_shape=jax.ShapeDtypeStruct((N_PAD, D_H), jnp.float32),
    )(acc, g1, dinv, b1, w2)


def _pool_body(acc_ref, g_ref, dinv_ref, b_ref, batch_ref, wl_ref, bl_ref,
               o_ref, sums_scr, cnt_scr):
    i = pl.program_id(0)

    @pl.when(i == 0)
    def _():
        sums_scr[...] = jnp.zeros_like(sums_scr)
        cnt_scr[...] = jnp.zeros_like(cnt_scr)

    dv = dinv_ref[...]
    h2 = jnp.maximum(dv * (acc_ref[0] + acc_ref[1] + g_ref[...]) + b_ref[...], 0.0)
    p = (batch_ref[...] == lax.broadcasted_iota(jnp.int32, (BLK, NG), 1))
    p = p.astype(jnp.float32)
    sums_scr[...] += lax.dot_general(p, h2, (((0,), (0,)), ((), ())),
                                     preferred_element_type=jnp.float32)
    cnt_scr[...] += lax.dot_general(p, jnp.ones((BLK, 1), jnp.float32),
                                    (((0,), (0,)), ((), ())),
                                    preferred_element_type=jnp.float32)

    @pl.when(i == pl.num_programs(0) - 1)
    def _():
        pooled = sums_scr[...] / jnp.maximum(cnt_scr[...], 1.0)
        o_ref[...] = jnp.dot(pooled, wl_ref[...],
                             preferred_element_type=jnp.float32) + bl_ref[...]


def _pool(acc, g2, dinv, b2, batch_pad, wlin, blin):
    return pl.pallas_call(
        _pool_body,
        grid=(N_PAD // BLK,),
        in_specs=[pl.BlockSpec((NC, BLK, D_H), lambda i: (0, i, 0)),
                  pl.BlockSpec((BLK, D_H), lambda i: (i, 0)),
                  pl.BlockSpec((BLK, 1), lambda i: (i, 0)),
                  pl.BlockSpec((1, D_H), lambda i: (0, 0)),
                  pl.BlockSpec((BLK, 1), lambda i: (i, 0)),
                  pl.BlockSpec((D_H, 2), lambda i: (0, 0)),
                  pl.BlockSpec((1, 2), lambda i: (0, 0))],
        out_specs=pl.BlockSpec((NG, 2), lambda i: (0, 0)),
        out_shape=jax.ShapeDtypeStruct((NG, 2), jnp.float32),
        scratch_shapes=[pltpu.VMEM((NG, D_H), jnp.float32),
                        pltpu.VMEM((NG, 1), jnp.float32)],
    )(acc, g2, dinv, b2, batch_pad, wlin, blin)


def kernel(x, edge_index, batch, W1, b1, W2, b2, Wlin, blin):
    src = edge_index[0].astype(jnp.int32)
    dst = edge_index[1].astype(jnp.int32)
    pad_e = E_PAD - E
    pad_ids = jnp.full((pad_e,), N, jnp.int32)
    src_l = jnp.concatenate([src, pad_ids]).reshape(NW, NCHUNK, CHUNK)
    dst_l = jnp.concatenate([dst, pad_ids]).reshape(NW, NCHUNK, CHUNK)
    x_pad = jnp.concatenate([x, jnp.zeros((N_PAD - N, D_IN), x.dtype)])
    batch_pad = jnp.concatenate(
        [batch.astype(jnp.int32), jnp.full((N_PAD - N,), NG, jnp.int32)]
    ).reshape(N_PAD, 1)

    deg_part = _sc_degree(dst_l)
    h1 = _matmul(x_pad, W1)
    g1, dinv = _scale(h1, deg_part.reshape(NC, N_PAD, 1))
    acc1 = _sc_scatter(g1, src_l, dst_l)
    g2 = _layer2(acc1, g1, dinv, b1.reshape(1, D_H), W2)
    acc2 = _sc_scatter(g2, src_l, dst_l)
    return _pool(acc2, g2, dinv, b2.reshape(1, D_H), batch_pad,
                 Wlin, blin.reshape(1, 2))


# speedup vs baseline: 19.7916x; 1.0141x over previous
import functools

import jax
import jax.numpy as jnp
from jax import lax
from jax.experimental import pallas as pl
from jax.experimental.pallas import tpu as pltpu
from jax.experimental.pallas import tpu_sc as plsc

N = 10000
E = 320000
D_IN = 128
D_H = 64
NG = 64

NC = 2
NS = 16
L = 16
NW = NC * NS
CHUNK = 128
NCHUNK = 80
E_PAD = NW * NCHUNK * CHUNK
N_PAD = 10240
RPT = N_PAD // NS
RING = 8
BLK = 1024

_mesh = plsc.VectorSubcoreMesh(core_axis_name="c", subcore_axis_name="s")
_sc_params = pltpu.CompilerParams(use_tc_tiling_on_sc=False)


@functools.partial(
    pl.kernel,
    out_type=jax.ShapeDtypeStruct((NC, N_PAD), jnp.float32),
    mesh=_mesh,
    scratch_types=[
        pltpu.VMEM_SHARED((N_PAD,), jnp.float32),
        pltpu.VMEM((NCHUNK, CHUNK), jnp.int32),
        pltpu.VMEM((CHUNK,), jnp.float32),
        pltpu.VMEM((RPT,), jnp.float32),
    ],
    compiler_params=_sc_params,
)
def _sc_degree(dst_hbm, deg_out, deg_sh, idx_v, ones_v, zero_v):
    c = lax.axis_index("c")
    s = lax.axis_index("s")
    wid = c * NS + s
    for i in range(CHUNK // L):
        ones_v[pl.ds(i * L, L)] = jnp.full((L,), 1.0, jnp.float32)
    for i in range(RPT // L):
        zero_v[pl.ds(i * L, L)] = jnp.zeros((L,), jnp.float32)
    pltpu.sync_copy(zero_v, deg_sh.at[pl.ds(s * RPT, RPT)])
    pltpu.sync_copy(dst_hbm.at[wid], idx_v)
    plsc.subcore_barrier()

    def body(j, carry):
        pltpu.sync_copy(ones_v, deg_sh.at[idx_v.at[j]], add=True)
        return carry

    lax.fori_loop(0, NCHUNK, body, 0)
    plsc.subcore_barrier()
    pltpu.sync_copy(deg_sh.at[pl.ds(s * RPT, RPT)],
                    deg_out.at[c, pl.ds(s * RPT, RPT)])


@functools.partial(
    pl.kernel,
    out_type=jax.ShapeDtypeStruct((NC, N_PAD, D_H), jnp.float32),
    mesh=_mesh,
    scratch_types=[
        pltpu.VMEM_SHARED((N_PAD, D_H), jnp.float32),
        pltpu.VMEM((NCHUNK, CHUNK), jnp.int32),
        pltpu.VMEM((NCHUNK, CHUNK), jnp.int32),
        pltpu.VMEM((RING, CHUNK, D_H), jnp.float32),
        pltpu.VMEM((64, D_H), jnp.float32),
        pltpu.SemaphoreType.DMA((RING,)),
        pltpu.SemaphoreType.DMA((RING,)),
    ],
    compiler_params=_sc_params,
)
def _sc_scatter(g_hbm, src_hbm, dst_hbm, acc_out,
                acc_sh, src_v, dst_v, rows, zero_v, sem_g, sem_s):
    c = lax.axis_index("c")
    s = lax.axis_index("s")
    wid = c * NS + s
    for i in range(64):
        for k in range(D_H // L):
            zero_v[i, pl.ds(k * L, L)] = jnp.zeros((L,), jnp.float32)
    for k in range(RPT // 64):
        pltpu.sync_copy(zero_v, acc_sh.at[pl.ds(s * RPT + k * 64, 64)])
    pltpu.sync_copy(src_hbm.at[wid], src_v)
    pltpu.sync_copy(dst_hbm.at[wid], dst_v)
    plsc.subcore_barrier()
    for k in range(RING):
        pltpu.async_copy(g_hbm.at[src_v.at[k]], rows.at[k], sem_g.at[k])

    def body(t, carry):
        base = RING * t
        for k in range(RING):
            pltpu.make_async_copy(g_hbm.at[src_v.at[base + k]],
                                  rows.at[k], sem_g.at[k]).wait()
            pltpu.async_copy(rows.at[k], acc_sh.at[dst_v.at[base + k]],
                             sem_s.at[k], add=True)
        for k in range(RING):
            pltpu.make_async_copy(rows.at[k], acc_sh.at[dst_v.at[base + k]],
                                  sem_s.at[k]).wait()

            @pl.when(base + k + RING < NCHUNK)
            def _():
                pltpu.async_copy(g_hbm.at[src_v.at[base + k + RING]],
                                 rows.at[k], sem_g.at[k])

        return carry

    lax.fori_loop(0, NCHUNK // RING, body, 0)
    plsc.subcore_barrier()
    pltpu.sync_copy(acc_sh.at[pl.ds(s * RPT, RPT)],
                    acc_out.at[c, pl.ds(s * RPT, RPT)])


def _mm_body(x_ref, w_ref, o_ref):
    o_ref[...] = jnp.dot(x_ref[...], w_ref[...], preferred_element_type=jnp.float32)


def _matmul(x, w):
    m, k = x.shape
    n = w.shape[1]
    return pl.pallas_call(
        _mm_body,
        grid=(m // BLK,),
        in_specs=[pl.BlockSpec((BLK, k), lambda i: (i, 0)),
                  pl.BlockSpec((k, n), lambda i: (0, 0))],
        out_specs=pl.BlockSpec((BLK, n), lambda i: (i, 0)),
        out_shape=jax.ShapeDtypeStruct((m, n), jnp.float32),
    )(x, w)


def _scale_body(h_ref, deg_ref, g_ref, dinv_ref):
    dv = lax.rsqrt(deg_ref[0] + deg_ref[1] + 1.0)
    dinv_ref[...] = dv
    g_ref[...] = h_ref[...] * dv


def _scale(h, deg_part):
    return pl.pallas_call(
        _scale_body,
        grid=(N_PAD // BLK,),
        in_specs=[pl.BlockSpec((BLK, D_H), lambda i: (i, 0)),
                  pl.BlockSpec((NC, BLK, 1), lambda i: (0, i, 0))],
        out_specs=[pl.BlockSpec((BLK, D_H), lambda i: (i, 0)),
                   pl.BlockSpec((BLK, 1), lambda i: (i, 0))],
        out_shape=[jax.ShapeDtypeStruct((N_PAD, D_H), jnp.float32),
                   jax.ShapeDtypeStruct((N_PAD, 1), jnp.float32)],
    )(h, deg_part)


def _layer2_body(acc_ref, g_ref, dinv_ref, b_ref, w_ref, o_ref):
    dv = dinv_ref[...]
    h1 = jnp.maximum(dv * (acc_ref[0] + acc_ref[1] + g_ref[...]) + b_ref[...], 0.0)
    o_ref[...] = jnp.dot(h1, w_ref[...], preferred_element_type=jnp.float32) * dv


def _layer2(acc, g1, dinv, b1, w2):
    return pl.pallas_call(
        _layer2_body,
        grid=(N_PAD // BLK,),
        in_specs=[pl.BlockSpec((NC, BLK, D_H), lambda i: (0, i, 0)),
                  pl.BlockSpec((BLK, D_H), lambda i: (i, 0)),
                  pl.BlockSpec((BLK, 1), lambda i: (i, 0)),
                  pl.BlockSpec((1, D_H), lambda i: (0, 0)),
                  pl.BlockSpec((D_H, D_H), lambda i: (0, 0))],
        out_specs=pl.BlockSpec((BLK, D_H), lambda i: (i, 0)),
        out_shape=jax.ShapeDtypeStruct((N_PAD, D_H), jnp.float32),
    )(acc, g1, dinv, b1, w2)


def _pool_body(acc_ref, g_ref, dinv_ref, b_ref, batch_ref, wl_ref, bl_ref,
               o_ref, sums_scr, cnt_scr):
    i = pl.program_id(0)

    @pl.when(i == 0)
    def _():
        sums_scr[...] = jnp.zeros_like(sums_scr)
        cnt_scr[...] = jnp.zeros_like(cnt_scr)

    dv = dinv_ref[...]
    h2 = jnp.maximum(dv * (acc_ref[0] + acc_ref[1] + g_ref[...]) + b_ref[...], 0.0)
    p = (batch_ref[...] == lax.broadcasted_iota(jnp.int32, (BLK, NG), 1))
    p = p.astype(jnp.float32)
    sums_scr[...] += lax.dot_general(p, h2, (((0,), (0,)), ((), ())),
                                     preferred_element_type=jnp.float32)
    cnt_scr[...] += lax.dot_general(p, jnp.ones((BLK, 1), jnp.float32),
                                    (((0,), (0,)), ((), ())),
                                    preferred_element_type=jnp.float32)

    @pl.when(i == pl.num_programs(0) - 1)
    def _():
        pooled = sums_scr[...] / jnp.maximum(cnt_scr[...], 1.0)
        o_ref[...] = jnp.dot(pooled, wl_ref[...],
                             preferred_element_type=jnp.float32) + bl_ref[...]


def _pool(acc, g2, dinv, b2, batch_pad, wlin, blin):
    return pl.pallas_call(
        _pool_body,
        grid=(N_PAD // BLK,),
        in_specs=[pl.BlockSpec((NC, BLK, D_H), lambda i: (0, i, 0)),
                  pl.BlockSpec((BLK, D_H), lambda i: (i, 0)),
                  pl.BlockSpec((BLK, 1), lambda i: (i, 0)),
                  pl.BlockSpec((1, D_H), lambda i: (0, 0)),
                  pl.BlockSpec((BLK, 1), lambda i: (i, 0)),
                  pl.BlockSpec((D_H, 2), lambda i: (0, 0)),
                  pl.BlockSpec((1, 2), lambda i: (0, 0))],
        out_specs=pl.BlockSpec((NG, 2), lambda i: (0, 0)),
        out_shape=jax.ShapeDtypeStruct((NG, 2), jnp.float32),
        scratch_shapes=[pltpu.VMEM((NG, D_H), jnp.float32),
                        pltpu.VMEM((NG, 1), jnp.float32)],
    )(acc, g2, dinv, b2, batch_pad, wlin, blin)


def kernel(x, edge_index, batch, W1, b1, W2, b2, Wlin, blin):
    src = edge_index[0].astype(jnp.int32)
    dst = edge_index[1].astype(jnp.int32)
    pad_e = E_PAD - E
    pad_ids = jnp.full((pad_e,), N, jnp.int32)
    src_l = jnp.concatenate([src, pad_ids]).reshape(NW, NCHUNK, CHUNK)
    dst_l = jnp.concatenate([dst, pad_ids]).reshape(NW, NCHUNK, CHUNK)
    x_pad = jnp.concatenate([x, jnp.zeros((N_PAD - N, D_IN), x.dtype)])
    batch_pad = jnp.concatenate(
        [batch.astype(jnp.int32), jnp.full((N_PAD - N,), NG, jnp.int32)]
    ).reshape(N_PAD, 1)

    deg_part = _sc_degree(dst_l)
    h1 = _matmul(x_pad, W1)
    g1, dinv = _scale(h1, deg_part.reshape(NC, N_PAD, 1))
    acc1 = _sc_scatter(g1, src_l, dst_l)
    g2 = _layer2(acc1, g1, dinv, b1.reshape(1, D_H), W2)
    acc2 = _sc_scatter(g2, src_l, dst_l)
    return _pool(acc2, g2, dinv, b2.reshape(1, D_H), batch_pad,
                 Wlin, blin.reshape(1, 2))


# speedup vs baseline: 41.9442x; 2.1193x over previous
import functools

import jax
import jax.numpy as jnp
from jax import lax
from jax.experimental import pallas as pl
from jax.experimental.pallas import tpu as pltpu
from jax.experimental.pallas import tpu_sc as plsc

N = 10000
E = 320000
D_IN = 128
D_H = 64
NG = 64

NC = 2
NS = 16
L = 16
NW = NC * NS
CHUNK = 128
NCHUNK = 80
E_PAD = NW * NCHUNK * CHUNK
N_PAD = 10240
RPT = N_PAD // NS
RING = 8
BLK = 1024

_mesh = plsc.VectorSubcoreMesh(core_axis_name="c", subcore_axis_name="s")
_sc_params = pltpu.CompilerParams(use_tc_tiling_on_sc=False)


@functools.partial(
    pl.kernel,
    out_type=jax.ShapeDtypeStruct((NC, N_PAD), jnp.float32),
    mesh=_mesh,
    scratch_types=[
        pltpu.VMEM_SHARED((N_PAD,), jnp.float32),
        pltpu.VMEM((NCHUNK, CHUNK), jnp.int32),
        pltpu.VMEM((CHUNK,), jnp.float32),
        pltpu.VMEM((RPT,), jnp.float32),
    ],
    compiler_params=_sc_params,
)
def _sc_degree(dst_hbm, deg_out, deg_sh, idx_v, ones_v, zero_v):
    c = lax.axis_index("c")
    s = lax.axis_index("s")
    wid = c * NS + s
    for i in range(CHUNK // L):
        ones_v[pl.ds(i * L, L)] = jnp.full((L,), 1.0, jnp.float32)
    for i in range(RPT // L):
        zero_v[pl.ds(i * L, L)] = jnp.zeros((L,), jnp.float32)
    pltpu.sync_copy(zero_v, deg_sh.at[pl.ds(s * RPT, RPT)])
    pltpu.sync_copy(dst_hbm.at[wid], idx_v)
    plsc.subcore_barrier()

    def body(j, carry):
        pltpu.sync_copy(ones_v, deg_sh.at[idx_v.at[j]], add=True)
        return carry

    lax.fori_loop(0, NCHUNK, body, 0)
    plsc.subcore_barrier()
    pltpu.sync_copy(deg_sh.at[pl.ds(s * RPT, RPT)],
                    deg_out.at[c, pl.ds(s * RPT, RPT)])


@functools.partial(
    pl.kernel,
    out_type=jax.ShapeDtypeStruct((NC, N_PAD, D_H), jnp.float32),
    mesh=_mesh,
    scratch_types=[
        pltpu.VMEM_SHARED((N_PAD, D_H), jnp.float32),
        pltpu.VMEM((NCHUNK, CHUNK), jnp.int32),
        pltpu.VMEM((NCHUNK, CHUNK), jnp.int32),
        pltpu.VMEM((RING, CHUNK, D_H), jnp.float32),
        pltpu.VMEM((64, D_H), jnp.float32),
        pltpu.SemaphoreType.DMA((RING,)),
        pltpu.SemaphoreType.DMA((RING,)),
    ],
    compiler_params=_sc_params,
)
def _sc_scatter(g_hbm, src_hbm, dst_hbm, acc_out,
                acc_sh, src_v, dst_v, rows, zero_v, sem_g, sem_s):
    c = lax.axis_index("c")
    s = lax.axis_index("s")
    wid = c * NS + s
    for i in range(64):
        for k in range(D_H // L):
            zero_v[i, pl.ds(k * L, L)] = jnp.zeros((L,), jnp.float32)
    for k in range(RPT // 64):
        pltpu.sync_copy(zero_v, acc_sh.at[pl.ds(s * RPT + k * 64, 64)])
    pltpu.sync_copy(src_hbm.at[wid], src_v)
    pltpu.sync_copy(dst_hbm.at[wid], dst_v)
    plsc.subcore_barrier()
    for k in range(RING):
        pltpu.async_copy(g_hbm.at[src_v.at[k]], rows.at[k], sem_g.at[k])

    def body(t, carry):
        base = RING * t
        for k in range(RING):
            pltpu.make_async_copy(g_hbm.at[src_v.at[base + k]],
                                  rows.at[k], sem_g.at[k]).wait()
            pltpu.async_copy(rows.at[k], acc_sh.at[dst_v.at[base + k]],
                             sem_s.at[k], add=True)
        for k in range(RING):
            pltpu.make_async_copy(rows.at[k], acc_sh.at[dst_v.at[base + k]],
                                  sem_s.at[k]).wait()

            @pl.when(base + k + RING < NCHUNK)
            def _():
                pltpu.async_copy(g_hbm.at[src_v.at[base + k + RING]],
                                 rows.at[k], sem_g.at[k])

        return carry

    lax.fori_loop(0, NCHUNK // RING, body, 0)
    plsc.subcore_barrier()
    pltpu.sync_copy(acc_sh.at[pl.ds(s * RPT, RPT)],
                    acc_out.at[c, pl.ds(s * RPT, RPT)])


def _mm_body(x_ref, w_ref, o_ref):
    o_ref[...] = jnp.dot(x_ref[...], w_ref[...], preferred_element_type=jnp.float32)


def _matmul(x, w):
    m, k = x.shape
    n = w.shape[1]
    return pl.pallas_call(
        _mm_body,
        grid=(m // BLK,),
        in_specs=[pl.BlockSpec((BLK, k), lambda i: (i, 0)),
                  pl.BlockSpec((k, n), lambda i: (0, 0))],
        out_specs=pl.BlockSpec((BLK, n), lambda i: (i, 0)),
        out_shape=jax.ShapeDtypeStruct((m, n), jnp.float32),
    )(x, w)


def _scale_body(h_ref, deg_ref, g_ref, dinv_ref):
    dv = lax.rsqrt(deg_ref[0] + deg_ref[1] + 1.0)
    dinv_ref[...] = dv
    g_ref[...] = h_ref[...] * dv


def _scale(h, deg_part):
    return pl.pallas_call(
        _scale_body,
        grid=(N_PAD // BLK,),
        in_specs=[pl.BlockSpec((BLK, D_H), lambda i: (i, 0)),
                  pl.BlockSpec((NC, BLK, 1), lambda i: (0, i, 0))],
        out_specs=[pl.BlockSpec((BLK, D_H), lambda i: (i, 0)),
                   pl.BlockSpec((BLK, 1), lambda i: (i, 0))],
        out_shape=[jax.ShapeDtypeStruct((N_PAD, D_H), jnp.float32),
                   jax.ShapeDtypeStruct((N_PAD, 1), jnp.float32)],
    )(h, deg_part)


def _layer2_body(acc_ref, g_ref, dinv_ref, b_ref, w_ref, o_ref):
    dv = dinv_ref[...]
    h1 = jnp.maximum(dv * (acc_ref[0] + acc_ref[1] + g_ref[...]) + b_ref[...], 0.0)
    o_ref[...] = jnp.dot(h1, w_ref[...], preferred_element_type=jnp.float32) * dv


def _layer2(acc, g1, dinv, b1, w2):
    return pl.pallas_call(
        _layer2_body,
        grid=(N_PAD // BLK,),
        in_specs=[pl.BlockSpec((NC, BLK, D_H), lambda i: (0, i, 0)),
                  pl.BlockSpec((BLK, D_H), lambda i: (i, 0)),
                  pl.BlockSpec((BLK, 1), lambda i: (i, 0)),
                  pl.BlockSpec((1, D_H), lambda i: (0, 0)),
                  pl.BlockSpec((D_H, D_H), lambda i: (0, 0))],
        out_specs=pl.BlockSpec((BLK, D_H), lambda i: (i, 0)),
        out_shape=jax.ShapeDtypeStruct((N_PAD, D_H), jnp.float32),
    )(acc, g1, dinv, b1, w2)


def _pool_body(acc_ref, g_ref, dinv_ref, b_ref, batch_ref, wl_ref, bl_ref,
               o_ref, sums_scr, cnt_scr):
    i = pl.program_id(0)

    @pl.when(i == 0)
    def _():
        sums_scr[...] = jnp.zeros_like(sums_scr)
        cnt_scr[...] = jnp.zeros_like(cnt_scr)

    dv = dinv_ref[...]
    h2 = jnp.maximum(dv * (acc_ref[0] + acc_ref[1] + g_ref[...]) + b_ref[...], 0.0)
    p = (batch_ref[...] == lax.broadcasted_iota(jnp.int32, (BLK, NG), 1))
    p = p.astype(jnp.float32)
    sums_scr[...] += lax.dot_general(p, h2, (((0,), (0,)), ((), ())),
                                     preferred_element_type=jnp.float32)
    cnt_scr[...] += lax.dot_general(p, jnp.ones((BLK, 1), jnp.float32),
                                    (((0,), (0,)), ((), ())),
                                    preferred_element_type=jnp.float32)

    @pl.when(i == pl.num_programs(0) - 1)
    def _():
        pooled = sums_scr[...] / jnp.maximum(cnt_scr[...], 1.0)
        o_ref[...] = jnp.dot(pooled, wl_ref[...],
                             preferred_element_type=jnp.float32) + bl_ref[...]


def _pool(acc, g2, dinv, b2, batch_pad, wlin, blin):
    return pl.pallas_call(
        _pool_body,
        grid=(N_PAD // BLK,),
        in_specs=[pl.BlockSpec((NC, BLK, D_H), lambda i: (0, i, 0)),
                  pl.BlockSpec((BLK, D_H), lambda i: (i, 0)),
                  pl.BlockSpec((BLK, 1), lambda i: (i, 0)),
                  pl.BlockSpec((1, D_H), lambda i: (0, 0)),
                  pl.BlockSpec((BLK, 1), lambda i: (i, 0)),
                  pl.BlockSpec((D_H, 2), lambda i: (0, 0)),
                  pl.BlockSpec((1, 2), lambda i: (0, 0))],
        out_specs=pl.BlockSpec((NG, 2), lambda i: (0, 0)),
        out_shape=jax.ShapeDtypeStruct((NG, 2), jnp.float32),
        scratch_shapes=[pltpu.VMEM((NG, D_H), jnp.float32),
                        pltpu.VMEM((NG, 1), jnp.float32)],
    )(acc, g2, dinv, b2, batch_pad, wlin, blin)


def kernel(x, edge_index, batch, W1, b1, W2, b2, Wlin, blin):
    src = edge_index[0].astype(jnp.int32)
    dst = edge_index[1].astype(jnp.int32)
    pad_e = E_PAD - E
    pad_ids = N + jnp.arange(pad_e, dtype=jnp.int32) % (N_PAD - N)
    src_l = jnp.concatenate([src, pad_ids]).reshape(NW, NCHUNK, CHUNK)
    dst_l = jnp.concatenate([dst, pad_ids]).reshape(NW, NCHUNK, CHUNK)
    x_pad = jnp.concatenate([x, jnp.zeros((N_PAD - N, D_IN), x.dtype)])
    batch_pad = jnp.concatenate(
        [batch.astype(jnp.int32), jnp.full((N_PAD - N,), NG, jnp.int32)]
    ).reshape(N_PAD, 1)

    deg_part = _sc_degree(dst_l)
    h1 = _matmul(x_pad, W1)
    g1, dinv = _scale(h1, deg_part.reshape(NC, N_PAD, 1))
    acc1 = _sc_scatter(g1, src_l, dst_l)
    g2 = _layer2(acc1, g1, dinv, b1.reshape(1, D_H), W2)
    acc2 = _sc_scatter(g2, src_l, dst_l)
    return _pool(acc2, g2, dinv, b2.reshape(1, D_H), batch_pad,
                 Wlin, blin.reshape(1, 2))


# speedup vs baseline: 48.8639x; 1.1650x over previous
import functools

import jax
import jax.numpy as jnp
from jax import lax
from jax.experimental import pallas as pl
from jax.experimental.pallas import tpu as pltpu
from jax.experimental.pallas import tpu_sc as plsc

N = 10000
E = 320000
D_IN = 128
D_H = 64
NG = 64

NC = 2
NS = 16
L = 16
NW = NC * NS
CHUNK = 128
NCHUNK = 80
E_PAD = NW * NCHUNK * CHUNK
N_PAD = 10240
RPT = N_PAD // NS
RING = 8
BLK = 1024

_mesh = plsc.VectorSubcoreMesh(core_axis_name="c", subcore_axis_name="s")
_sc_params = pltpu.CompilerParams(use_tc_tiling_on_sc=False)


@functools.partial(
    pl.kernel,
    out_type=jax.ShapeDtypeStruct((NC, N_PAD), jnp.float32),
    mesh=_mesh,
    scratch_types=[
        pltpu.VMEM_SHARED((N_PAD,), jnp.float32),
        pltpu.VMEM((NCHUNK, CHUNK), jnp.int32),
        pltpu.VMEM((CHUNK,), jnp.float32),
        pltpu.VMEM((RPT,), jnp.float32),
    ],
    compiler_params=_sc_params,
)
def _sc_degree(dst_hbm, deg_out, deg_sh, idx_v, ones_v, zero_v):
    c = lax.axis_index("c")
    s = lax.axis_index("s")
    wid = c * NS + s
    for i in range(CHUNK // L):
        ones_v[pl.ds(i * L, L)] = jnp.full((L,), 1.0, jnp.float32)
    for i in range(RPT // L):
        zero_v[pl.ds(i * L, L)] = jnp.zeros((L,), jnp.float32)
    pltpu.sync_copy(zero_v, deg_sh.at[pl.ds(s * RPT, RPT)])
    pltpu.sync_copy(dst_hbm.at[wid], idx_v)
    plsc.subcore_barrier()

    def body(j, carry):
        pltpu.sync_copy(ones_v, deg_sh.at[idx_v.at[j]], add=True)
        return carry

    lax.fori_loop(0, NCHUNK, body, 0)
    plsc.subcore_barrier()
    pltpu.sync_copy(deg_sh.at[pl.ds(s * RPT, RPT)],
                    deg_out.at[c, pl.ds(s * RPT, RPT)])


@functools.partial(
    pl.kernel,
    out_type=jax.ShapeDtypeStruct((N_PAD, NC * D_H), jnp.float32),
    mesh=_mesh,
    scratch_types=[
        pltpu.VMEM_SHARED((N_PAD, D_H), jnp.float32),
        pltpu.VMEM((NCHUNK, CHUNK), jnp.int32),
        pltpu.VMEM((NCHUNK, CHUNK), jnp.int32),
        pltpu.VMEM((RING, CHUNK, D_H), jnp.float32),
        pltpu.VMEM((64, D_H), jnp.float32),
        pltpu.SemaphoreType.DMA((RING,)),
        pltpu.SemaphoreType.DMA((RING,)),
    ],
    compiler_params=_sc_params,
)
def _sc_scatter(g_hbm, src_hbm, dst_hbm, acc_out,
                acc_sh, src_v, dst_v, rows, zero_v, sem_g, sem_s):
    c = lax.axis_index("c")
    s = lax.axis_index("s")
    wid = c * NS + s
    for i in range(64):
        for k in range(D_H // L):
            zero_v[i, pl.ds(k * L, L)] = jnp.zeros((L,), jnp.float32)
    for k in range(RPT // 64):
        pltpu.sync_copy(zero_v, acc_sh.at[pl.ds(s * RPT + k * 64, 64)])
    pltpu.sync_copy(src_hbm.at[wid], src_v)
    pltpu.sync_copy(dst_hbm.at[wid], dst_v)
    plsc.subcore_barrier()
    for k in range(RING):
        pltpu.async_copy(g_hbm.at[src_v.at[k]], rows.at[k], sem_g.at[k])

    def body(t, carry):
        base = RING * t
        for k in range(RING):
            pltpu.make_async_copy(g_hbm.at[src_v.at[base + k]],
                                  rows.at[k], sem_g.at[k]).wait()
            pltpu.async_copy(rows.at[k], acc_sh.at[dst_v.at[base + k]],
                             sem_s.at[k], add=True)
        for k in range(RING):
            pltpu.make_async_copy(rows.at[k], acc_sh.at[dst_v.at[base + k]],
                                  sem_s.at[k]).wait()

            @pl.when(base + k + RING < NCHUNK)
            def _():
                pltpu.async_copy(g_hbm.at[src_v.at[base + k + RING]],
                                 rows.at[k], sem_g.at[k])

        return carry

    lax.fori_loop(0, NCHUNK // RING, body, 0)
    plsc.subcore_barrier()
    pltpu.sync_copy(acc_sh.at[pl.ds(s * RPT, RPT)],
                    acc_out.at[pl.ds(s * RPT, RPT), pl.ds(c * D_H, D_H)])


def _mm_body(x_ref, w_ref, o_ref):
    o_ref[...] = jnp.dot(x_ref[...], w_ref[...], preferred_element_type=jnp.float32)


def _matmul(x, w):
    m, k = x.shape
    n = w.shape[1]
    return pl.pallas_call(
        _mm_body,
        grid=(m // BLK,),
        in_specs=[pl.BlockSpec((BLK, k), lambda i: (i, 0)),
                  pl.BlockSpec((k, n), lambda i: (0, 0))],
        out_specs=pl.BlockSpec((BLK, n), lambda i: (i, 0)),
        out_shape=jax.ShapeDtypeStruct((m, n), jnp.float32),
    )(x, w)


def _dinv_col(deg_ref):
    dv = lax.rsqrt(deg_ref[0:1, :] + deg_ref[1:2, :] + 1.0)
    return jnp.transpose(dv, (1, 0))


def _scale_body(h_ref, deg_ref, g_ref):
    g_ref[...] = h_ref[...] * _dinv_col(deg_ref)


def _scale(h, deg_part):
    return pl.pallas_call(
        _scale_body,
        grid=(N_PAD // BLK,),
        in_specs=[pl.BlockSpec((BLK, D_H), lambda i: (i, 0)),
                  pl.BlockSpec((NC, BLK), lambda i: (0, i))],
        out_specs=pl.BlockSpec((BLK, D_H), lambda i: (i, 0)),
        out_shape=jax.ShapeDtypeStruct((N_PAD, D_H), jnp.float32),
    )(h, deg_part)


def _layer2_body(acc_ref, g_ref, deg_ref, b_ref, w_ref, o_ref):
    dv = _dinv_col(deg_ref)
    accsum = acc_ref[:, :D_H] + acc_ref[:, D_H:]
    h1 = jnp.maximum(dv * (accsum + g_ref[...]) + b_ref[...], 0.0)
    o_ref[...] = jnp.dot(h1, w_ref[...], preferred_element_type=jnp.float32) * dv


def _layer2(acc, g1, deg_part, b1, w2):
    return pl.pallas_call(
        _layer2_body,
        grid=(N_PAD // BLK,),
        in_specs=[pl.BlockSpec((BLK, NC * D_H), lambda i: (i, 0)),
                  pl.BlockSpec((BLK, D_H), lambda i: (i, 0)),
                  pl.BlockSpec((NC, BLK), lambda i: (0, i)),
                  pl.BlockSpec((1, D_H), lambda i: (0, 0)),
                  pl.BlockSpec((D_H, D_H), lambda i: (0, 0))],
        out_specs=pl.BlockSpec((BLK, D_H), lambda i: (i, 0)),
        out_shape=jax.ShapeDtypeStruct((N_PAD, D_H), jnp.float32),
    )(acc, g1, deg_part, b1, w2)


def _pool_body(acc_ref, g_ref, deg_ref, b_ref, batch_ref, wl_ref, bl_ref,
               o_ref, sums_scr, cnt_scr):
    i = pl.program_id(0)

    @pl.when(i == 0)
    def _():
        sums_scr[...] = jnp.zeros_like(sums_scr)
        cnt_scr[...] = jnp.zeros_like(cnt_scr)

    dv = _dinv_col(deg_ref)
    accsum = acc_ref[:, :D_H] + acc_ref[:, D_H:]
    h2 = jnp.maximum(dv * (accsum + g_ref[...]) + b_ref[...], 0.0)
    pt = (batch_ref[...] == lax.broadcasted_iota(jnp.int32, (NG, BLK), 0))
    pt = pt.astype(jnp.float32)
    sums_scr[...] += lax.dot_general(pt, h2, (((1,), (0,)), ((), ())),
                                     preferred_element_type=jnp.float32)
    cnt_scr[...] += lax.dot_general(pt, jnp.ones((BLK, 1), jnp.float32),
                                    (((1,), (0,)), ((), ())),
                                    preferred_element_type=jnp.float32)

    @pl.when(i == pl.num_programs(0) - 1)
    def _():
        pooled = sums_scr[...] / jnp.maximum(cnt_scr[...], 1.0)
        o_ref[...] = jnp.dot(pooled, wl_ref[...],
                             preferred_element_type=jnp.float32) + bl_ref[...]


def _pool(acc, g2, deg_part, b2, batch_row, wlin, blin):
    return pl.pallas_call(
        _pool_body,
        grid=(N_PAD // BLK,),
        in_specs=[pl.BlockSpec((BLK, NC * D_H), lambda i: (i, 0)),
                  pl.BlockSpec((BLK, D_H), lambda i: (i, 0)),
                  pl.BlockSpec((NC, BLK), lambda i: (0, i)),
                  pl.BlockSpec((1, D_H), lambda i: (0, 0)),
                  pl.BlockSpec((1, BLK), lambda i: (0, i)),
                  pl.BlockSpec((D_H, 2), lambda i: (0, 0)),
                  pl.BlockSpec((1, 2), lambda i: (0, 0))],
        out_specs=pl.BlockSpec((NG, 2), lambda i: (0, 0)),
        out_shape=jax.ShapeDtypeStruct((NG, 2), jnp.float32),
        scratch_shapes=[pltpu.VMEM((NG, D_H), jnp.float32),
                        pltpu.VMEM((NG, 1), jnp.float32)],
    )(acc, g2, deg_part, b2, batch_row, wlin, blin)


def kernel(x, edge_index, batch, W1, b1, W2, b2, Wlin, blin):
    src = edge_index[0].astype(jnp.int32)
    dst = edge_index[1].astype(jnp.int32)
    pad_e = E_PAD - E
    pad_ids = N + jnp.arange(pad_e, dtype=jnp.int32) % (N_PAD - N)
    src_l = jnp.concatenate([src, pad_ids]).reshape(NW, NCHUNK, CHUNK)
    dst_l = jnp.concatenate([dst, pad_ids]).reshape(NW, NCHUNK, CHUNK)
    x_pad = jnp.concatenate([x, jnp.zeros((N_PAD - N, D_IN), x.dtype)])
    batch_row = jnp.concatenate(
        [batch.astype(jnp.int32), jnp.full((N_PAD - N,), NG, jnp.int32)]
    ).reshape(1, N_PAD)

    deg_part = _sc_degree(dst_l)
    h1 = _matmul(x_pad, W1)
    g1 = _scale(h1, deg_part)
    acc1 = _sc_scatter(g1, src_l, dst_l)
    g2 = _layer2(acc1, g1, deg_part, b1.reshape(1, D_H), W2)
    acc2 = _sc_scatter(g2, src_l, dst_l)
    return _pool(acc2, g2, deg_part, b2.reshape(1, D_H), batch_row,
                 Wlin, blin.reshape(1, 2))
